# baseline (speedup 1.0000x reference)
import functools

import jax
import jax.numpy as jnp
from jax import lax
from jax.experimental import pallas as pl
from jax.experimental.pallas import tpu as pltpu
from jax.experimental.pallas import tpu_sc as plsc

N_NODES = 10000
N_EDGES = 320000
D = 128
FC_IN = 16
FC_HID = 64

NC = 2
NS = 16
NW = NC * NS
EPW = N_EDGES // NW
CB = 80
NCHUNK = EPW // CB
N_PAD = 10240
RPS = N_PAD // NS

_INV_SQRT_IN = 1.0 / (FC_IN ** 0.5)
_INV_SQRT_HID = 1.0 / (FC_HID ** 0.5)
_INV_SQRT_D = 1.0 / (D ** 0.5)


def _edge_weight_body(es_ref, attr_ref, w1_ref, w2_ref, out_ref):
    h = jnp.dot(es_ref[...], w1_ref[...], preferred_element_type=jnp.float32)
    h = jax.nn.silu(h * _INV_SQRT_IN)
    w = jnp.dot(h, w2_ref[...], preferred_element_type=jnp.float32)
    out_ref[...] = w * _INV_SQRT_HID * attr_ref[...]


def _edge_weights(edge_scalars, edge_attr_sh, W_fc1, W_fc2):
    EB = 6400
    grid = N_EDGES // EB
    return pl.pallas_call(
        _edge_weight_body,
        grid=(grid,),
        in_specs=[
            pl.BlockSpec((EB, FC_IN), lambda i: (i, 0)),
            pl.BlockSpec((EB, 1), lambda i: (i, 0)),
            pl.BlockSpec((FC_IN, FC_HID), lambda i: (0, 0)),
            pl.BlockSpec((FC_HID, D), lambda i: (0, 0)),
        ],
        out_specs=pl.BlockSpec((EB, D), lambda i: (i, 0)),
        out_shape=jax.ShapeDtypeStruct((N_EDGES, D), jnp.float32),
    )(edge_scalars, edge_attr_sh, W_fc1, W_fc2)


G5 = 5


def _sc_body(src_hbm, dst_hbm, dst5_hbm, w_hbm, x_hbm, iota_hbm,
             feat_out, cnt_out,
             src_idx, dst_idx, dst5, rows0, rows1, wbuf0, wbuf1,
             acc_sh,
             sem, sem0, sem1, ssem0, ssem1):
    cid = lax.axis_index("c")
    sid = lax.axis_index("s")
    wid = cid * NS + sid

    def _fill(buf, val):
        def _f(r, _):
            for c8 in range(D // 16):
                buf[r, pl.ds(c8 * 16, 16)] = jnp.full((16,), val, jnp.float32)
            return 0
        lax.fori_loop(0, CB, _f, 0)

    def _zinit(q, _):
        off = sid * RPS + q * CB
        pltpu.sync_copy(iota_hbm.at[pl.ds(off, CB)], src_idx.at[0])
        pltpu.sync_copy(rows0, acc_sh.at[src_idx.at[0]])
        return 0

    def _wout(out_ref):
        def _w(q, _):
            off = sid * RPS + q * CB
            pltpu.sync_copy(iota_hbm.at[pl.ds(off, CB)], src_idx.at[0])
            pltpu.async_copy(acc_sh.at[src_idx.at[0]], rows0, sem).wait()
            pltpu.sync_copy(rows0, out_ref.at[cid, pl.ds(off, CB)])
            return 0
        lax.fori_loop(0, RPS // CB, _w, 0)

    _fill(rows0, 0.0)
    lax.fori_loop(0, RPS // CB, _zinit, 0)
    plsc.subcore_barrier()

    def _issue(jj, b, rbuf, wb, sg):
        pltpu.async_copy(x_hbm.at[src_idx.at[b]], rbuf, sg)
        pltpu.async_copy(w_hbm.at[pl.ds(wid * EPW + jj * CB, CB), :], wb, sg)

    def _drain(rbuf, wb, sg):
        pltpu.make_async_copy(x_hbm.at[src_idx.at[0]], rbuf, sg).wait()
        pltpu.make_async_copy(w_hbm.at[pl.ds(0, CB), :], wb, sg).wait()

    def _mul(rbuf, wb):
        def _m(r, _):
            for c8 in range(D // 16):
                sl = pl.ds(c8 * 16, 16)
                rbuf[r, sl] = rbuf[r, sl] * wb[r, sl]
            return 0
        lax.fori_loop(0, CB, _m, 0)

    def _sdrain(rbuf, b, ss):
        pltpu.make_async_copy(rbuf, acc_sh.at[dst_idx.at[b]], ss).wait()

    pltpu.sync_copy(src_hbm.at[wid, 0], src_idx.at[0])
    pltpu.sync_copy(src_hbm.at[wid, 1], src_idx.at[1])
    pltpu.sync_copy(dst_hbm.at[wid, 0], dst_idx.at[0])
    _issue(0, 0, rows0, wbuf0, sem0)

    def _body(j, b, nb, rb, wb, rnb, wnb, sb, snb, ssb, ssnb):
        @pl.when(j >= 1)
        def _():
            _sdrain(rnb, nb, ssnb)
        @pl.when(j + 1 < NCHUNK)
        def _():
            _issue(j + 1, nb, rnb, wnb, snb)
            pltpu.sync_copy(dst_hbm.at[wid, j + 1], dst_idx.at[nb])
        _drain(rb, wb, sb)
        @pl.when(j + 2 < NCHUNK)
        def _():
            pltpu.sync_copy(src_hbm.at[wid, j + 2], src_idx.at[b])
        _mul(rb, wb)
        pltpu.async_copy(rb, acc_sh.at[dst_idx.at[b]], ssb, add=True)

    def _chunk(j, _):
        even = (j % 2) == 0

        @pl.when(even)
        def _():
            _body(j, 0, 1, rows0, wbuf0, rows1, wbuf1, sem0, sem1, ssem0, ssem1)

        @pl.when(jnp.logical_not(even))
        def _():
            _body(j, 1, 0, rows1, wbuf1, rows0, wbuf0, sem1, sem0, ssem1, ssem0)

        return 0

    lax.fori_loop(0, NCHUNK, _chunk, 0)
    _sdrain(rows0, 0, ssem0)
    plsc.subcore_barrier()
    _wout(feat_out)
    plsc.subcore_barrier()

    _fill(rows0, 0.0)
    lax.fori_loop(0, RPS // CB, _zinit, 0)
    plsc.subcore_barrier()
    _fill(rows0, 1.0)

    def _cgroup(g, _):
        pltpu.sync_copy(dst5_hbm.at[wid, g], dst5)
        for u in range(G5):
            pltpu.async_copy(rows0, acc_sh.at[dst5.at[u]], sem0, add=True)
        for u in range(G5):
            pltpu.make_async_copy(w_hbm.at[pl.ds(0, CB), :], rows0, sem0).wait()
        return 0

    lax.fori_loop(0, NCHUNK // G5, _cgroup, 0)
    plsc.subcore_barrier()
    _wout(cnt_out)


_sc_scatter = functools.partial(
    pl.kernel,
    mesh=plsc.VectorSubcoreMesh(core_axis_name="c", subcore_axis_name="s"),
    out_type=[
        jax.ShapeDtypeStruct((NC, N_PAD, D), jnp.float32),
        jax.ShapeDtypeStruct((NC, N_PAD, D), jnp.float32),
    ],
    scratch_types=[
        pltpu.VMEM((2, CB), jnp.int32),
        pltpu.VMEM((2, CB), jnp.int32),
        pltpu.VMEM((G5, CB), jnp.int32),
        pltpu.VMEM((CB, D), jnp.float32),
        pltpu.VMEM((CB, D), jnp.float32),
        pltpu.VMEM((CB, D), jnp.float32),
        pltpu.VMEM((CB, D), jnp.float32),
        pltpu.VMEM_SHARED((N_PAD, D), jnp.float32),
        pltpu.SemaphoreType.DMA,
        pltpu.SemaphoreType.DMA,
        pltpu.SemaphoreType.DMA,
        pltpu.SemaphoreType.DMA,
        pltpu.SemaphoreType.DMA,
    ],
)(_sc_body)


def _finish_body(feat_ref, cnt_ref, x_ref, wsc_ref, wlin_ref, wa_ref, out_ref):
    s = feat_ref[0] + feat_ref[1]
    c = cnt_ref[0, :, 0:1] + cnt_ref[1, :, 0:1]
    nf = s / jnp.maximum(c, 1.0)
    nsc = jnp.dot(x_ref[...], wsc_ref[...], preferred_element_type=jnp.float32)
    ncv = jnp.dot(nf, wlin_ref[...], preferred_element_type=jnp.float32)
    alpha = jnp.sum(nf * wa_ref[...], axis=1, keepdims=True) * _INV_SQRT_D
    out_ref[...] = (nsc + alpha * ncv) * _INV_SQRT_D


def _finish(feat_p, cnt_p, x, W_sc, W_lin, W_alpha):
    NB = 2000
    grid = N_NODES // NB
    return pl.pallas_call(
        _finish_body,
        grid=(grid,),
        in_specs=[
            pl.BlockSpec((NC, NB, D), lambda i: (0, i, 0)),
            pl.BlockSpec((NC, NB, D), lambda i: (0, i, 0)),
            pl.BlockSpec((NB, D), lambda i: (i, 0)),
            pl.BlockSpec((D, D), lambda i: (0, 0)),
            pl.BlockSpec((D, D), lambda i: (0, 0)),
            pl.BlockSpec((1, D), lambda i: (0, 0)),
        ],
        out_specs=pl.BlockSpec((NB, D), lambda i: (i, 0)),
        out_shape=jax.ShapeDtypeStruct((N_NODES, D), jnp.float32),
    )(feat_p, cnt_p, x, W_sc, W_lin, W_alpha.reshape(1, D))


def kernel(x, edge_index, edge_attr_sh, edge_scalars, W_sc, W_fc1, W_fc2, W_lin, W_alpha):
    w_eff = _edge_weights(edge_scalars, edge_attr_sh, W_fc1, W_fc2)
    iota = jnp.arange(N_PAD, dtype=jnp.int32)
    src3 = edge_index[0].reshape(NW, NCHUNK, CB)
    dst3 = edge_index[1].reshape(NW, NCHUNK, CB)
    dst5 = edge_index[1].reshape(NW, NCHUNK // G5, G5, CB)
    feat_p, cnt_p = _sc_scatter(src3, dst3, dst5, w_eff, x, iota)
    return _finish(feat_p, cnt_p, x, W_sc, W_lin, W_alpha)


# speedup vs baseline: 1.0148x; 1.0148x over previous
import functools

import jax
import jax.numpy as jnp
from jax import lax
from jax.experimental import pallas as pl
from jax.experimental.pallas import tpu as pltpu
from jax.experimental.pallas import tpu_sc as plsc

N_NODES = 10000
N_EDGES = 320000
D = 128
FC_IN = 16
FC_HID = 64

NC = 2
NS = 16
NW = NC * NS
EPW = N_EDGES // NW
CB = 80
NCHUNK = EPW // CB
N_PAD = 10240
RPS = N_PAD // NS

_INV_SQRT_IN = 1.0 / (FC_IN ** 0.5)
_INV_SQRT_HID = 1.0 / (FC_HID ** 0.5)
_INV_SQRT_D = 1.0 / (D ** 0.5)


def _edge_weight_body(es_ref, attr_ref, w1_ref, w2_ref, out_ref):
    h = jnp.dot(es_ref[...], w1_ref[...], preferred_element_type=jnp.float32)
    h = jax.nn.silu(h * _INV_SQRT_IN)
    w = jnp.dot(h, w2_ref[...], preferred_element_type=jnp.float32)
    out_ref[...] = w * _INV_SQRT_HID * attr_ref[...]


def _edge_weights(edge_scalars, edge_attr_sh, W_fc1, W_fc2):
    EB = 6400
    grid = N_EDGES // EB
    return pl.pallas_call(
        _edge_weight_body,
        grid=(grid,),
        in_specs=[
            pl.BlockSpec((EB, FC_IN), lambda i: (i, 0)),
            pl.BlockSpec((EB, 1), lambda i: (i, 0)),
            pl.BlockSpec((FC_IN, FC_HID), lambda i: (0, 0)),
            pl.BlockSpec((FC_HID, D), lambda i: (0, 0)),
        ],
        out_specs=pl.BlockSpec((EB, D), lambda i: (i, 0)),
        out_shape=jax.ShapeDtypeStruct((N_EDGES, D), jnp.float32),
    )(edge_scalars, edge_attr_sh, W_fc1, W_fc2)


G5 = 5


def _sc_body(src_hbm, dst_hbm, dst5_hbm, w_hbm, x_hbm, iota_hbm,
             feat_out, cnt_out,
             src_idx, dst_idx, dst5, rows0, rows1, wbuf0, wbuf1,
             acc_sh,
             sem, sem0, sem1, ssem0, ssem1):
    cid = lax.axis_index("c")
    sid = lax.axis_index("s")
    wid = cid * NS + sid

    def _fill(buf, val):
        def _f(r, _):
            for c8 in range(D // 16):
                buf[r, pl.ds(c8 * 16, 16)] = jnp.full((16,), val, jnp.float32)
            return 0
        lax.fori_loop(0, CB, _f, 0)

    def _zinit(q, _):
        off = sid * RPS + q * CB
        pltpu.sync_copy(iota_hbm.at[pl.ds(off, CB)], src_idx.at[0])
        pltpu.sync_copy(rows0, acc_sh.at[src_idx.at[0]])
        return 0

    def _wout(out_ref):
        def _w(q, _):
            off = sid * RPS + q * CB
            pltpu.sync_copy(iota_hbm.at[pl.ds(off, CB)], src_idx.at[0])
            pltpu.async_copy(acc_sh.at[src_idx.at[0]], rows0, sem).wait()
            pltpu.sync_copy(rows0, out_ref.at[cid, pl.ds(off, CB)])
            return 0
        lax.fori_loop(0, RPS // CB, _w, 0)

    _fill(rows0, 0.0)
    lax.fori_loop(0, RPS // CB, _zinit, 0)
    plsc.subcore_barrier()

    def _issue(jj, b, rbuf, wb, sg):
        pltpu.sync_copy(src_hbm.at[wid, jj], src_idx.at[b])
        pltpu.sync_copy(dst_hbm.at[wid, jj], dst_idx.at[b])
        pltpu.async_copy(x_hbm.at[src_idx.at[b]], rbuf, sg)
        pltpu.async_copy(w_hbm.at[pl.ds(wid * EPW + jj * CB, CB), :], wb, sg)

    def _drain(rbuf, wb, sg):
        pltpu.make_async_copy(x_hbm.at[src_idx.at[0]], rbuf, sg).wait()
        pltpu.make_async_copy(w_hbm.at[pl.ds(0, CB), :], wb, sg).wait()

    def _mul(rbuf, wb):
        def _m(r, _):
            for c8 in range(D // 16):
                sl = pl.ds(c8 * 16, 16)
                rbuf[r, sl] = rbuf[r, sl] * wb[r, sl]
            return 0
        lax.fori_loop(0, CB, _m, 0)

    _issue(0, 0, rows0, wbuf0, sem0)

    def _sdrain(rbuf, b, ss):
        pltpu.make_async_copy(rbuf, acc_sh.at[dst_idx.at[b]], ss).wait()

    def _chunk(j, _):
        even = (j % 2) == 0

        @pl.when(even)
        def _():
            @pl.when(j >= 1)
            def _():
                _sdrain(rows1, 1, ssem1)
            @pl.when(j + 1 < NCHUNK)
            def _():
                _issue(j + 1, 1, rows1, wbuf1, sem1)
            _drain(rows0, wbuf0, sem0)
            _mul(rows0, wbuf0)
            pltpu.async_copy(rows0, acc_sh.at[dst_idx.at[0]], ssem0, add=True)

        @pl.when(jnp.logical_not(even))
        def _():
            _sdrain(rows0, 0, ssem0)
            @pl.when(j + 1 < NCHUNK)
            def _():
                _issue(j + 1, 0, rows0, wbuf0, sem0)
            _drain(rows1, wbuf1, sem1)
            _mul(rows1, wbuf1)
            pltpu.async_copy(rows1, acc_sh.at[dst_idx.at[1]], ssem1, add=True)

        return 0

    lax.fori_loop(0, NCHUNK, _chunk, 0)
    _sdrain(rows0, 0, ssem0)
    plsc.subcore_barrier()
    _wout(feat_out)
    plsc.subcore_barrier()

    _fill(rows0, 0.0)
    lax.fori_loop(0, RPS // CB, _zinit, 0)
    plsc.subcore_barrier()
    _fill(rows0, 1.0)

    def _cgroup(g, _):
        pltpu.sync_copy(dst5_hbm.at[wid, g], dst5)
        for u in range(G5):
            pltpu.async_copy(rows0, acc_sh.at[dst5.at[u]], sem0, add=True)
        for u in range(G5):
            pltpu.make_async_copy(w_hbm.at[pl.ds(0, CB), :], rows0, sem0).wait()
        return 0

    lax.fori_loop(0, NCHUNK // G5, _cgroup, 0)
    plsc.subcore_barrier()
    _wout(cnt_out)


_sc_scatter = functools.partial(
    pl.kernel,
    mesh=plsc.VectorSubcoreMesh(core_axis_name="c", subcore_axis_name="s"),
    out_type=[
        jax.ShapeDtypeStruct((NC, N_PAD, D), jnp.float32),
        jax.ShapeDtypeStruct((NC, N_PAD, D), jnp.float32),
    ],
    scratch_types=[
        pltpu.VMEM((2, CB), jnp.int32),
        pltpu.VMEM((2, CB), jnp.int32),
        pltpu.VMEM((G5, CB), jnp.int32),
        pltpu.VMEM((CB, D), jnp.float32),
        pltpu.VMEM((CB, D), jnp.float32),
        pltpu.VMEM((CB, D), jnp.float32),
        pltpu.VMEM((CB, D), jnp.float32),
        pltpu.VMEM_SHARED((N_PAD, D), jnp.float32),
        pltpu.SemaphoreType.DMA,
        pltpu.SemaphoreType.DMA,
        pltpu.SemaphoreType.DMA,
        pltpu.SemaphoreType.DMA,
        pltpu.SemaphoreType.DMA,
    ],
)(_sc_body)


def _finish_body(feat_ref, cnt_ref, x_ref, wsc_ref, wlin_ref, wa_ref, out_ref):
    s = feat_ref[0] + feat_ref[1]
    c = cnt_ref[0, :, 0:1] + cnt_ref[1, :, 0:1]
    nf = s / jnp.maximum(c, 1.0)
    nsc = jnp.dot(x_ref[...], wsc_ref[...], preferred_element_type=jnp.float32)
    ncv = jnp.dot(nf, wlin_ref[...], preferred_element_type=jnp.float32)
    alpha = jnp.sum(nf * wa_ref[...], axis=1, keepdims=True) * _INV_SQRT_D
    out_ref[...] = (nsc + alpha * ncv) * _INV_SQRT_D


def _finish(feat_p, cnt_p, x, W_sc, W_lin, W_alpha):
    NB = 2000
    grid = N_NODES // NB
    return pl.pallas_call(
        _finish_body,
        grid=(grid,),
        in_specs=[
            pl.BlockSpec((NC, NB, D), lambda i: (0, i, 0)),
            pl.BlockSpec((NC, NB, D), lambda i: (0, i, 0)),
            pl.BlockSpec((NB, D), lambda i: (i, 0)),
            pl.BlockSpec((D, D), lambda i: (0, 0)),
            pl.BlockSpec((D, D), lambda i: (0, 0)),
            pl.BlockSpec((1, D), lambda i: (0, 0)),
        ],
        out_specs=pl.BlockSpec((NB, D), lambda i: (i, 0)),
        out_shape=jax.ShapeDtypeStruct((N_NODES, D), jnp.float32),
    )(feat_p, cnt_p, x, W_sc, W_lin, W_alpha.reshape(1, D))


def kernel(x, edge_index, edge_attr_sh, edge_scalars, W_sc, W_fc1, W_fc2, W_lin, W_alpha):
    w_eff = _edge_weights(edge_scalars, edge_attr_sh, W_fc1, W_fc2)
    iota = jnp.arange(N_PAD, dtype=jnp.int32)
    src3 = edge_index[0].reshape(NW, NCHUNK, CB)
    dst3 = edge_index[1].reshape(NW, NCHUNK, CB)
    dst5 = edge_index[1].reshape(NW, NCHUNK // G5, G5, CB)
    feat_p, cnt_p = _sc_scatter(src3, dst3, dst5, w_eff, x, iota)
    return _finish(feat_p, cnt_p, x, W_sc, W_lin, W_alpha)
